# baseline (speedup 1.0000x reference)
import functools

import jax
import jax.numpy as jnp
from jax import lax
from jax.experimental import pallas as pl
from jax.experimental.pallas import tpu as pltpu
from jax.experimental.pallas import tpu_sc as plsc

N_USERS = 100000
N_ITEMS = 100000
EMB = 64
HID = 128
P = 3
B = 4096
D = P * HID

NC = 2
NS = 16
NW = NC * NS
BPW = B // NW

BB = 512
GRID = B // BB


def _sc_gather(user_table, item_table, users, pos_items, neg_items):
    mesh = plsc.VectorSubcoreMesh(core_axis_name="c", subcore_axis_name="s")
    out = jax.ShapeDtypeStruct((B, D), jnp.float32)

    @functools.partial(
        pl.kernel,
        mesh=mesh,
        out_type=(out, out, out),
        scratch_types=[
            pltpu.VMEM((BPW,), jnp.int32),
            pltpu.VMEM((BPW,), jnp.int32),
            pltpu.VMEM((BPW,), jnp.int32),
            pltpu.VMEM((BPW, D), jnp.float32),
            pltpu.SemaphoreType.DMA,
        ],
    )
    def gather_kernel(ut_hbm, it_hbm, ui_hbm, pi_hbm, ni_hbm,
                      u_out, p_out, n_out,
                      uidx_v, pidx_v, nidx_v, buf, gsem):
        wid = lax.axis_index("s") * NC + lax.axis_index("c")
        sl = pl.ds(wid * BPW, BPW)
        pltpu.sync_copy(ui_hbm.at[sl], uidx_v)
        pltpu.sync_copy(pi_hbm.at[sl], pidx_v)
        pltpu.sync_copy(ni_hbm.at[sl], nidx_v)
        pltpu.async_copy(ut_hbm.at[uidx_v], buf, gsem).wait()
        pltpu.sync_copy(buf, u_out.at[sl])
        pltpu.async_copy(it_hbm.at[pidx_v], buf, gsem).wait()
        pltpu.sync_copy(buf, p_out.at[sl])
        pltpu.async_copy(it_hbm.at[nidx_v], buf, gsem).wait()
        pltpu.sync_copy(buf, n_out.at[sl])

    return gather_kernel(user_table, item_table, users, pos_items, neg_items)


def _attn_math(u, pg, ng, wu, wi, a):
    dot = lambda x, y: jax.lax.dot(
        x, y, precision=jax.lax.Precision.HIGHEST,
        preferred_element_type=jnp.float32)
    PU = [dot(u[:, k * HID:(k + 1) * HID], wu) for k in range(P)]
    MU = [dot(PU[k], a) for k in range(P)]
    PPos = [dot(pg[:, k * HID:(k + 1) * HID], wi) for k in range(P)]
    PNeg = [dot(ng[:, k * HID:(k + 1) * HID], wi) for k in range(P)]

    def max3(v0, v1, v2):
        return jnp.maximum(jnp.maximum(v0, v1), v2)

    def soft3(v):
        m = max3(v[0], v[1], v[2])
        e = [jnp.exp(x - m) for x in v]
        r = 1.0 / (e[0] + e[1] + e[2])
        return [x * r for x in e]

    def pair(PI):
        M = [[jnp.sum(MU[p] * PI[q], axis=1, keepdims=True)
              for q in range(P)] for p in range(P)]
        u_att = soft3([max3(M[p][0], M[p][1], M[p][2]) for p in range(P)])
        i_att = soft3([max3(M[0][q], M[1][q], M[2][q]) for q in range(P)])
        att_u = u_att[0] * PU[0] + u_att[1] * PU[1] + u_att[2] * PU[2]
        att_i = i_att[0] * PI[0] + i_att[1] * PI[1] + i_att[2] * PI[2]
        return att_u, att_i

    pu_att, pi_att = pair(PPos)
    nu_att, ni_att = pair(PNeg)
    return pu_att, pi_att, nu_att, ni_att


def _attn_body(u_ref, p_ref, n_ref, wu_ref, wi_ref, a_ref,
               pu_ref, pi_ref, nu_ref, ni_ref):
    pu, pi, nu, ni = _attn_math(u_ref[...], p_ref[...], n_ref[...],
                                wu_ref[...], wi_ref[...], a_ref[...])
    pu_ref[...] = pu
    pi_ref[...] = pi
    nu_ref[...] = nu
    ni_ref[...] = ni


def _tc_attention(u_g, p_g, n_g, W_u, W_i, A):
    out = jax.ShapeDtypeStruct((B, EMB), jnp.float32)
    row_spec = pl.BlockSpec((BB, D), lambda i: (i, 0))
    full = lambda s: pl.BlockSpec(s, lambda i: (0, 0))
    return pl.pallas_call(
        _attn_body,
        grid=(GRID,),
        in_specs=[row_spec, row_spec, row_spec,
                  full((HID, EMB)), full((HID, EMB)), full((EMB, EMB))],
        out_specs=[pl.BlockSpec((BB, EMB), lambda i: (i, 0))] * 4,
        out_shape=(out, out, out, out),
    )(u_g, p_g, n_g, W_u, W_i, A)


def kernel(users, pos_items, neg_items, multi_user_embed, multi_item_embed,
           W_u, W_i, A):
    ut = multi_user_embed.reshape(N_USERS, D)
    it = multi_item_embed.reshape(N_ITEMS, D)
    u_g, p_g, n_g = _sc_gather(ut, it,
                               users.astype(jnp.int32),
                               pos_items.astype(jnp.int32),
                               neg_items.astype(jnp.int32))
    return _tc_attention(u_g, p_g, n_g, W_u, W_i, A)
